# baseline (speedup 1.0000x reference)
import functools

import jax
import jax.numpy as jnp
from jax import lax
from jax.experimental import pallas as pl
from jax.experimental.pallas import tpu as pltpu
from jax.experimental.pallas import tpu_sc as plsc

N = 10000
NPAD = 10240
E = 320000
EPAD = 327680
DF = 128
DH = 128
DC = 16
NC, NS, L = 2, 16, 16
NW = NC * NS
CH = 128
EW = EPAD // NW
NCH = EW // CH
NCHG = 16
NGRP = NCH // NCHG
RPT = NPAD // NS


def _mesh():
    return plsc.VectorSubcoreMesh(
        core_axis_name="c", subcore_axis_name="s",
        num_cores=NC, num_subcores=NS)


def _deg_body(col_hbm, w_hbm, out_hbm, col_v, w_v, zb, acc):
    c = lax.axis_index("c")
    s = lax.axis_index("s")
    wid = c * NS + s
    pltpu.sync_copy(col_hbm.at[pl.ds(wid * NCH, NCH)], col_v)
    pltpu.sync_copy(w_hbm.at[pl.ds(wid * NCH, NCH)], w_v)

    def zloop(i, carry):
        zb[pl.ds(i * L, L)] = jnp.zeros((L,), jnp.float32)
        return carry
    lax.fori_loop(0, RPT // L, zloop, 0)
    pltpu.sync_copy(zb, acc.at[pl.ds(s * RPT, RPT)])
    plsc.subcore_barrier()

    def chunk(j, carry):
        pltpu.sync_copy(w_v.at[j], acc.at[col_v.at[j]], add=True)
        return carry
    lax.fori_loop(0, NCH, chunk, 0)
    plsc.subcore_barrier()
    pltpu.sync_copy(acc.at[pl.ds(s * RPT, RPT)],
                    out_hbm.at[c, pl.ds(s * RPT, RPT)])


def _run_deg(col2d, w2d):
    k = pl.kernel(
        _deg_body,
        out_type=jax.ShapeDtypeStruct((NC, NPAD), jnp.float32),
        mesh=_mesh(),
        scratch_types=[
            pltpu.VMEM((NCH, CH), jnp.int32),
            pltpu.VMEM((NCH, CH), jnp.float32),
            pltpu.VMEM((RPT,), jnp.float32),
            pltpu.VMEM_SHARED((NPAD,), jnp.float32),
        ],
    )
    return k(col2d, w2d)


def _msg_body(DS, NH, *refs):
    g_halves = refs[:NH]
    row_hbm, col_hbm, wexp_hbm = refs[NH:NH + 3]
    outs = refs[NH + 3:2 * NH + 3]
    (row_v, col_v, gb0, gb1, gb2, gb3, wb0, wb1, wb2, wb3,
     sem_g, sem_s, sem_w, g_sh, acc) = refs[2 * NH + 3:]
    gbufs = [gb0, gb1, gb2, gb3]
    wbufs = [wb0, wb1, wb2, wb3]
    nv = DS // L
    c = lax.axis_index("c")
    s = lax.axis_index("s")
    wid = c * NS + s

    def scale(gbuf, wbuf):
        def srow(i, carry):
            wspl = wbuf[i, pl.ds(0, L)]
            for d in range(nv):
                gbuf[i, pl.ds(d * L, L)] = gbuf[i, pl.ds(d * L, L)] * wspl
            return carry
        lax.fori_loop(0, CH, srow, 0)

    for h in range(NH):
        def zrow(i, carry):
            for d in range(nv):
                gb0[i, pl.ds(d * L, L)] = jnp.zeros((L,), jnp.float32)
            return carry
        lax.fori_loop(0, CH, zrow, 0)
        for k in range(RPT // CH):
            pltpu.sync_copy(gb0, acc.at[pl.ds(s * RPT + k * CH, CH)])
        pltpu.sync_copy(g_halves[h].at[pl.ds(s * RPT, RPT)],
                        g_sh.at[pl.ds(s * RPT, RPT)])
        plsc.subcore_barrier()

        for grp in range(NGRP):
            gbase = wid * NCH + grp * NCHG
            pltpu.sync_copy(row_hbm.at[pl.ds(gbase, NCHG)], row_v)
            pltpu.sync_copy(col_hbm.at[pl.ds(gbase, NCHG)], col_v)
            for p in range(2):
                pltpu.async_copy(g_sh.at[row_v.at[p]], gbufs[p], sem_g)
                pltpu.async_copy(
                    wexp_hbm.at[pl.ds((gbase + p) * CH, CH)], wbufs[p], sem_w)

            def ring(t, carry):
                for b in range(4):
                    j = 4 * t + b
                    bn = (b + 2) % 4
                    pltpu.make_async_copy(
                        g_sh.at[row_v.at[j]], gbufs[b], sem_g).wait()
                    pltpu.make_async_copy(
                        wexp_hbm.at[pl.ds((gbase + j) * CH, CH)],
                        wbufs[b], sem_w).wait()
                    scale(gbufs[b], wbufs[b])
                    pltpu.async_copy(
                        gbufs[b], acc.at[col_v.at[j]], sem_s, add=True)

                    @pl.when(j >= 2)
                    def _():
                        pltpu.make_async_copy(
                            gbufs[bn], acc.at[col_v.at[j - 2]], sem_s).wait()

                    @pl.when(j + 2 < NCHG)
                    def _():
                        pltpu.async_copy(
                            g_sh.at[row_v.at[j + 2]], gbufs[bn], sem_g)
                        pltpu.async_copy(
                            wexp_hbm.at[pl.ds((gbase + j + 2) * CH, CH)],
                            wbufs[bn], sem_w)
                return carry
            lax.fori_loop(0, NCHG // 4, ring, 0)
            pltpu.make_async_copy(
                gbufs[(NCHG - 2) % 4],
                acc.at[col_v.at[NCHG - 2]], sem_s).wait()
            pltpu.make_async_copy(
                gbufs[(NCHG - 1) % 4],
                acc.at[col_v.at[NCHG - 1]], sem_s).wait()
        plsc.subcore_barrier()
        pltpu.sync_copy(acc.at[pl.ds(s * RPT, RPT)],
                        outs[h].at[c, pl.ds(s * RPT, RPT)])


def _run_msg(g_halves, row2d, col2d, wexp, DS):
    NH = len(g_halves)
    k = pl.kernel(
        functools.partial(_msg_body, DS, NH),
        out_type=[jax.ShapeDtypeStruct((NC, NPAD, DS), jnp.float32)] * NH,
        mesh=_mesh(),
        scratch_types=[
            pltpu.VMEM((NCHG, CH), jnp.int32),
            pltpu.VMEM((NCHG, CH), jnp.int32),
            pltpu.VMEM((CH, DS), jnp.float32),
            pltpu.VMEM((CH, DS), jnp.float32),
            pltpu.VMEM((CH, DS), jnp.float32),
            pltpu.VMEM((CH, DS), jnp.float32),
            pltpu.VMEM((CH, L), jnp.float32),
            pltpu.VMEM((CH, L), jnp.float32),
            pltpu.VMEM((CH, L), jnp.float32),
            pltpu.VMEM((CH, L), jnp.float32),
            pltpu.SemaphoreType.DMA,
            pltpu.SemaphoreType.DMA,
            pltpu.SemaphoreType.DMA,
            pltpu.VMEM_SHARED((NPAD, DS), jnp.float32),
            pltpu.VMEM_SHARED((NPAD, DS), jnp.float32),
        ],
        compiler_params=pltpu.CompilerParams(use_tc_tiling_on_sc=False),
    )
    return k(*g_halves, row2d, col2d, wexp)


def _dinv_of(deg_ref):
    deg = deg_ref[0] + deg_ref[1] + 1.0
    return jnp.where(deg > 0, lax.rsqrt(deg), 0.0)


def _tc1_body(deg_ref, x_ref, w1_ref, glo_ref, ghi_ref):
    dinv = _dinv_of(deg_ref)
    z = jnp.dot(x_ref[...], w1_ref[...], preferred_element_type=jnp.float32)
    g = z * dinv[:, None]
    glo_ref[...] = g[:, :DH // 2]
    ghi_ref[...] = g[:, DH // 2:]


def _tc2_body(deg_ref, slo_ref, shi_ref, glo_ref, ghi_ref, w2_ref, b1_ref,
              g2_ref):
    dinv = _dinv_of(deg_ref)
    s = jnp.concatenate(
        [slo_ref[0] + slo_ref[1] + glo_ref[...],
         shi_ref[0] + shi_ref[1] + ghi_ref[...]], axis=1)
    agg = s * dinv[:, None]
    h1 = jnp.maximum(agg + b1_ref[...][None, :], 0.0)
    z2 = jnp.dot(h1, w2_ref[...], preferred_element_type=jnp.float32)
    g2_ref[...] = z2 * dinv[:, None]


def _tc3_body(deg_ref, s_ref, g2_ref, b2_ref, out_ref):
    dinv = _dinv_of(deg_ref)
    out_ref[...] = ((s_ref[0] + s_ref[1] + g2_ref[...]) * dinv[:, None]
                    + b2_ref[...][None, :])


def _tc1(degp, xpad, W1):
    return pl.pallas_call(
        _tc1_body,
        out_shape=[jax.ShapeDtypeStruct((NPAD, DH // 2), jnp.float32)] * 2,
    )(degp, xpad, W1)


def _tc2(degp, s1lo, s1hi, g1lo, g1hi, W2, b1):
    return pl.pallas_call(
        _tc2_body,
        out_shape=jax.ShapeDtypeStruct((NPAD, DC), jnp.float32),
    )(degp, s1lo, s1hi, g1lo, g1hi, W2, b1)


def _tc3(degp, s2, g2, b2):
    return pl.pallas_call(
        _tc3_body,
        out_shape=jax.ShapeDtypeStruct((NPAD, DC), jnp.float32),
    )(degp, s2, g2, b2)


def kernel(x, edge_index, edge_weight, W1, b1, W2, b2):
    row2d = jnp.zeros((EPAD,), jnp.int32).at[:E].set(
        edge_index[0].astype(jnp.int32)).reshape(EPAD // CH, CH)
    col2d = jnp.zeros((EPAD,), jnp.int32).at[:E].set(
        edge_index[1].astype(jnp.int32)).reshape(EPAD // CH, CH)
    wpad = jnp.zeros((EPAD,), jnp.float32).at[:E].set(
        edge_weight.astype(jnp.float32))
    w2d = wpad.reshape(EPAD // CH, CH)
    wexp = jnp.broadcast_to(wpad[:, None], (EPAD, L))
    xpad = jnp.zeros((NPAD, DF), jnp.float32).at[:N].set(x)

    degp = _run_deg(col2d, w2d)
    g1lo, g1hi = _tc1(degp, xpad, W1)
    s1lo, s1hi = _run_msg([g1lo, g1hi], row2d, col2d, wexp, DH // 2)
    g2 = _tc2(degp, s1lo, s1hi, g1lo, g1hi, W2, b1)
    s2, = _run_msg([g2], row2d, col2d, wexp, DC)
    out = _tc3(degp, s2, g2, b2)
    return out[:N]


# speedup vs baseline: 1.1254x; 1.1254x over previous
import functools

import jax
import jax.numpy as jnp
from jax import lax
from jax.experimental import pallas as pl
from jax.experimental.pallas import tpu as pltpu
from jax.experimental.pallas import tpu_sc as plsc

N = 10000
NPAD = 10240
E = 320000
EPAD = 327680
DF = 128
DH = 128
DC = 16
NC, NS, L = 2, 16, 16
NW = NC * NS
CH = 128
EW = EPAD // NW
NCH = EW // CH
NCHG = 16
NGRP = NCH // NCHG
RPT = NPAD // NS


def _mesh():
    return plsc.VectorSubcoreMesh(
        core_axis_name="c", subcore_axis_name="s",
        num_cores=NC, num_subcores=NS)


def _deg_body(col_hbm, w_hbm, out_hbm, col_v, w_v, zb, acc):
    c = lax.axis_index("c")
    s = lax.axis_index("s")
    wid = c * NS + s
    pltpu.sync_copy(col_hbm.at[pl.ds(wid * NCH, NCH)], col_v)
    pltpu.sync_copy(w_hbm.at[pl.ds(wid * NCH, NCH)], w_v)

    def zloop(i, carry):
        zb[pl.ds(i * L, L)] = jnp.zeros((L,), jnp.float32)
        return carry
    lax.fori_loop(0, RPT // L, zloop, 0)
    pltpu.sync_copy(zb, acc.at[pl.ds(s * RPT, RPT)])
    plsc.subcore_barrier()

    def chunk(j, carry):
        pltpu.sync_copy(w_v.at[j], acc.at[col_v.at[j]], add=True)
        return carry
    lax.fori_loop(0, NCH, chunk, 0)
    plsc.subcore_barrier()
    pltpu.sync_copy(acc.at[pl.ds(s * RPT, RPT)],
                    out_hbm.at[c, pl.ds(s * RPT, RPT)])


def _run_deg(col2d, w2d):
    k = pl.kernel(
        _deg_body,
        out_type=jax.ShapeDtypeStruct((NC, NPAD), jnp.float32),
        mesh=_mesh(),
        scratch_types=[
            pltpu.VMEM((NCH, CH), jnp.int32),
            pltpu.VMEM((NCH, CH), jnp.float32),
            pltpu.VMEM((RPT,), jnp.float32),
            pltpu.VMEM_SHARED((NPAD,), jnp.float32),
        ],
    )
    return k(col2d, w2d)


def _msg_body(DS, NH, *refs):
    g_halves = refs[:NH]
    row_hbm, col_hbm, w_hbm = refs[NH:NH + 3]
    outs = refs[NH + 3:2 * NH + 3]
    (row_v, col_v, w_v, gb0, gb1, gb2, gb3,
     sem_g, sem_s, g_sh, acc) = refs[2 * NH + 3:]
    gbufs = [gb0, gb1, gb2, gb3]
    nv = DS // L
    c = lax.axis_index("c")
    s = lax.axis_index("s")
    wid = c * NS + s

    def scale(gbuf, j):
        def scale_group(g, gcarry):
            w16 = w_v[j, pl.ds(g * L, L)]
            for k in range(L):
                i = g * L + k
                wspl = jnp.full((L,), w16[k], jnp.float32)
                for d in range(nv):
                    gbuf[i, pl.ds(d * L, L)] = gbuf[i, pl.ds(d * L, L)] * wspl
            return gcarry
        lax.fori_loop(0, CH // L, scale_group, 0)

    for h in range(NH):
        def zrow(i, carry):
            for d in range(nv):
                gb0[i, pl.ds(d * L, L)] = jnp.zeros((L,), jnp.float32)
            return carry
        lax.fori_loop(0, CH, zrow, 0)
        for k in range(RPT // CH):
            pltpu.sync_copy(gb0, acc.at[pl.ds(s * RPT + k * CH, CH)])
        pltpu.sync_copy(g_halves[h].at[pl.ds(s * RPT, RPT)],
                        g_sh.at[pl.ds(s * RPT, RPT)])
        plsc.subcore_barrier()

        for grp in range(NGRP):
            gbase = wid * NCH + grp * NCHG
            pltpu.sync_copy(row_hbm.at[pl.ds(gbase, NCHG)], row_v)
            pltpu.sync_copy(col_hbm.at[pl.ds(gbase, NCHG)], col_v)
            pltpu.sync_copy(w_hbm.at[pl.ds(gbase, NCHG)], w_v)
            pltpu.async_copy(g_sh.at[row_v.at[0]], gbufs[0], sem_g)
            pltpu.async_copy(g_sh.at[row_v.at[1]], gbufs[1], sem_g)

            def ring(t, carry):
                for b in range(4):
                    j = 4 * t + b
                    bn = (b + 2) % 4
                    pltpu.make_async_copy(
                        g_sh.at[row_v.at[j]], gbufs[b], sem_g).wait()
                    scale(gbufs[b], j)
                    pltpu.async_copy(
                        gbufs[b], acc.at[col_v.at[j]], sem_s, add=True)

                    @pl.when(j >= 2)
                    def _():
                        pltpu.make_async_copy(
                            gbufs[bn], acc.at[col_v.at[j - 2]], sem_s).wait()

                    @pl.when(j + 2 < NCHG)
                    def _():
                        pltpu.async_copy(
                            g_sh.at[row_v.at[j + 2]], gbufs[bn], sem_g)
                return carry
            lax.fori_loop(0, NCHG // 4, ring, 0)
            pltpu.make_async_copy(
                gbufs[(NCHG - 2) % 4],
                acc.at[col_v.at[NCHG - 2]], sem_s).wait()
            pltpu.make_async_copy(
                gbufs[(NCHG - 1) % 4],
                acc.at[col_v.at[NCHG - 1]], sem_s).wait()
        plsc.subcore_barrier()
        pltpu.sync_copy(acc.at[pl.ds(s * RPT, RPT)],
                        outs[h].at[c, pl.ds(s * RPT, RPT)])


def _run_msg(g_halves, row2d, col2d, w2d, DS):
    NH = len(g_halves)
    k = pl.kernel(
        functools.partial(_msg_body, DS, NH),
        out_type=[jax.ShapeDtypeStruct((NC, NPAD, DS), jnp.float32)] * NH,
        mesh=_mesh(),
        scratch_types=[
            pltpu.VMEM((NCHG, CH), jnp.int32),
            pltpu.VMEM((NCHG, CH), jnp.int32),
            pltpu.VMEM((NCHG, CH), jnp.float32),
            pltpu.VMEM((CH, DS), jnp.float32),
            pltpu.VMEM((CH, DS), jnp.float32),
            pltpu.VMEM((CH, DS), jnp.float32),
            pltpu.VMEM((CH, DS), jnp.float32),
            pltpu.SemaphoreType.DMA,
            pltpu.SemaphoreType.DMA,
            pltpu.VMEM_SHARED((NPAD, DS), jnp.float32),
            pltpu.VMEM_SHARED((NPAD, DS), jnp.float32),
        ],
        compiler_params=pltpu.CompilerParams(use_tc_tiling_on_sc=False),
    )
    return k(*g_halves, row2d, col2d, w2d)


def _dinv_of(deg_ref):
    deg = deg_ref[0] + deg_ref[1] + 1.0
    return jnp.where(deg > 0, lax.rsqrt(deg), 0.0)


def _tc1_body(deg_ref, x_ref, w1_ref, glo_ref, ghi_ref):
    dinv = _dinv_of(deg_ref)
    z = jnp.dot(x_ref[...], w1_ref[...], preferred_element_type=jnp.float32)
    g = z * dinv[:, None]
    glo_ref[...] = g[:, :DH // 2]
    ghi_ref[...] = g[:, DH // 2:]


def _tc2_body(deg_ref, slo_ref, shi_ref, glo_ref, ghi_ref, w2_ref, b1_ref,
              g2_ref):
    dinv = _dinv_of(deg_ref)
    s = jnp.concatenate(
        [slo_ref[0] + slo_ref[1] + glo_ref[...],
         shi_ref[0] + shi_ref[1] + ghi_ref[...]], axis=1)
    agg = s * dinv[:, None]
    h1 = jnp.maximum(agg + b1_ref[...][None, :], 0.0)
    z2 = jnp.dot(h1, w2_ref[...], preferred_element_type=jnp.float32)
    g2_ref[...] = z2 * dinv[:, None]


def _tc3_body(deg_ref, s_ref, g2_ref, b2_ref, out_ref):
    dinv = _dinv_of(deg_ref)
    out_ref[...] = ((s_ref[0] + s_ref[1] + g2_ref[...]) * dinv[:, None]
                    + b2_ref[...][None, :])


def _tc1(degp, xpad, W1):
    return pl.pallas_call(
        _tc1_body,
        out_shape=[jax.ShapeDtypeStruct((NPAD, DH // 2), jnp.float32)] * 2,
    )(degp, xpad, W1)


def _tc2(degp, s1lo, s1hi, g1lo, g1hi, W2, b1):
    return pl.pallas_call(
        _tc2_body,
        out_shape=jax.ShapeDtypeStruct((NPAD, DC), jnp.float32),
    )(degp, s1lo, s1hi, g1lo, g1hi, W2, b1)


def _tc3(degp, s2, g2, b2):
    return pl.pallas_call(
        _tc3_body,
        out_shape=jax.ShapeDtypeStruct((NPAD, DC), jnp.float32),
    )(degp, s2, g2, b2)


def kernel(x, edge_index, edge_weight, W1, b1, W2, b2):
    row2d = jnp.zeros((EPAD,), jnp.int32).at[:E].set(
        edge_index[0].astype(jnp.int32)).reshape(EPAD // CH, CH)
    col2d = jnp.zeros((EPAD,), jnp.int32).at[:E].set(
        edge_index[1].astype(jnp.int32)).reshape(EPAD // CH, CH)
    w2d = jnp.zeros((EPAD,), jnp.float32).at[:E].set(
        edge_weight.astype(jnp.float32)).reshape(EPAD // CH, CH)
    xpad = jnp.zeros((NPAD, DF), jnp.float32).at[:N].set(x)

    degp = _run_deg(col2d, w2d)
    g1lo, g1hi = _tc1(degp, xpad, W1)
    s1lo, s1hi = _run_msg([g1lo, g1hi], row2d, col2d, w2d, DH // 2)
    g2 = _tc2(degp, s1lo, s1hi, g1lo, g1hi, W2, b1)
    s2, = _run_msg([g2], row2d, col2d, w2d, DC)
    out = _tc3(degp, s2, g2, b2)
    return out[:N]


# speedup vs baseline: 1.4638x; 1.3007x over previous
import functools

import jax
import jax.numpy as jnp
from jax import lax
from jax.experimental import pallas as pl
from jax.experimental.pallas import tpu as pltpu
from jax.experimental.pallas import tpu_sc as plsc

N = 10000
NPAD = 10240
E = 320000
EPAD = 327680
DF = 128
DH = 128
DC = 16
NC, NS, L = 2, 16, 16
NW = NC * NS
CH = 128
EW = EPAD // NW
NCH = EW // CH
NCHG = 16
NGRP = NCH // NCHG
RPT = NPAD // NS


def _mesh():
    return plsc.VectorSubcoreMesh(
        core_axis_name="c", subcore_axis_name="s",
        num_cores=NC, num_subcores=NS)


def _deg_body(col_hbm, w_hbm, out_hbm, col_v, w_v, zb, acc):
    c = lax.axis_index("c")
    s = lax.axis_index("s")
    wid = c * NS + s
    pltpu.sync_copy(col_hbm.at[pl.ds(wid * NCH, NCH)], col_v)
    pltpu.sync_copy(w_hbm.at[pl.ds(wid * NCH, NCH)], w_v)

    def zloop(i, carry):
        zb[pl.ds(i * L, L)] = jnp.zeros((L,), jnp.float32)
        return carry
    lax.fori_loop(0, RPT // L, zloop, 0)
    pltpu.sync_copy(zb, acc.at[pl.ds(s * RPT, RPT)])
    plsc.subcore_barrier()

    def chunk(j, carry):
        pltpu.sync_copy(w_v.at[j], acc.at[col_v.at[j]], add=True)
        return carry
    lax.fori_loop(0, NCH, chunk, 0)
    plsc.subcore_barrier()
    pltpu.sync_copy(acc.at[pl.ds(s * RPT, RPT)],
                    out_hbm.at[c, pl.ds(s * RPT, RPT)])


def _run_deg(col2d, w2d):
    k = pl.kernel(
        _deg_body,
        out_type=jax.ShapeDtypeStruct((NC, NPAD), jnp.float32),
        mesh=_mesh(),
        scratch_types=[
            pltpu.VMEM((NCH, CH), jnp.int32),
            pltpu.VMEM((NCH, CH), jnp.float32),
            pltpu.VMEM((RPT,), jnp.float32),
            pltpu.VMEM_SHARED((NPAD,), jnp.float32),
        ],
    )
    return k(col2d, w2d)


def _msg_body(DS, NH, *refs):
    g_halves = refs[:NH]
    row_hbm, col_hbm, w_hbm = refs[NH:NH + 3]
    outs = refs[NH + 3:2 * NH + 3]
    (row_v, col_v, w_v, gb0, gb1, gb2, gb3, wbuf,
     sem_g, sem_s, g_sh, acc) = refs[2 * NH + 3:]
    gbufs = [gb0, gb1, gb2, gb3]
    nv = DS // L
    c = lax.axis_index("c")
    s = lax.axis_index("s")
    wid = c * NS + s

    dn = lax.GatherDimensionNumbers(
        offset_dims=(), collapsed_slice_dims=(0,), start_index_map=(0,))
    splat_idx = [jnp.full((L, 1), k, jnp.int32) for k in range(L)]

    def scale(gbuf, wbuf, j):
        def bgrp(g, gcarry):
            w16 = w_v[j, pl.ds(g * L, L)]
            for k in range(L):
                wbuf[pl.ds((g * L + k) * L, L)] = lax.gather(
                    w16, splat_idx[k], dn, (1,),
                    mode=lax.GatherScatterMode.PROMISE_IN_BOUNDS)
            return gcarry
        lax.fori_loop(0, CH // L, bgrp, 0)

        def srow(i, icarry):
            wspl = wbuf[pl.ds(i * L, L)]
            for d in range(nv):
                gbuf[i, pl.ds(d * L, L)] = gbuf[i, pl.ds(d * L, L)] * wspl
            return icarry
        lax.fori_loop(0, CH, srow, 0)

    for h in range(NH):
        def zrow(i, carry):
            for d in range(nv):
                gb0[i, pl.ds(d * L, L)] = jnp.zeros((L,), jnp.float32)
            return carry
        lax.fori_loop(0, CH, zrow, 0)
        for k in range(RPT // CH):
            pltpu.sync_copy(gb0, acc.at[pl.ds(s * RPT + k * CH, CH)])
        pltpu.sync_copy(g_halves[h].at[pl.ds(s * RPT, RPT)],
                        g_sh.at[pl.ds(s * RPT, RPT)])
        plsc.subcore_barrier()

        for grp in range(NGRP):
            gbase = wid * NCH + grp * NCHG
            pltpu.sync_copy(row_hbm.at[pl.ds(gbase, NCHG)], row_v)
            pltpu.sync_copy(col_hbm.at[pl.ds(gbase, NCHG)], col_v)
            pltpu.sync_copy(w_hbm.at[pl.ds(gbase, NCHG)], w_v)
            pltpu.async_copy(g_sh.at[row_v.at[0]], gbufs[0], sem_g)
            pltpu.async_copy(g_sh.at[row_v.at[1]], gbufs[1], sem_g)

            def ring(t, carry):
                for b in range(4):
                    j = 4 * t + b
                    bn = (b + 2) % 4
                    pltpu.make_async_copy(
                        g_sh.at[row_v.at[j]], gbufs[b], sem_g).wait()
                    scale(gbufs[b], wbuf, j)
                    pltpu.async_copy(
                        gbufs[b], acc.at[col_v.at[j]], sem_s, add=True)

                    @pl.when(j >= 2)
                    def _():
                        pltpu.make_async_copy(
                            gbufs[bn], acc.at[col_v.at[j - 2]], sem_s).wait()

                    @pl.when(j + 2 < NCHG)
                    def _():
                        pltpu.async_copy(
                            g_sh.at[row_v.at[j + 2]], gbufs[bn], sem_g)
                return carry
            lax.fori_loop(0, NCHG // 4, ring, 0)
            pltpu.make_async_copy(
                gbufs[(NCHG - 2) % 4],
                acc.at[col_v.at[NCHG - 2]], sem_s).wait()
            pltpu.make_async_copy(
                gbufs[(NCHG - 1) % 4],
                acc.at[col_v.at[NCHG - 1]], sem_s).wait()
        plsc.subcore_barrier()
        pltpu.sync_copy(acc.at[pl.ds(s * RPT, RPT)],
                        outs[h].at[c, pl.ds(s * RPT, RPT)])


def _run_msg(g_halves, row2d, col2d, w2d, DS):
    NH = len(g_halves)
    k = pl.kernel(
        functools.partial(_msg_body, DS, NH),
        out_type=[jax.ShapeDtypeStruct((NC, NPAD, DS), jnp.float32)] * NH,
        mesh=_mesh(),
        scratch_types=[
            pltpu.VMEM((NCHG, CH), jnp.int32),
            pltpu.VMEM((NCHG, CH), jnp.int32),
            pltpu.VMEM((NCHG, CH), jnp.float32),
            pltpu.VMEM((CH, DS), jnp.float32),
            pltpu.VMEM((CH, DS), jnp.float32),
            pltpu.VMEM((CH, DS), jnp.float32),
            pltpu.VMEM((CH, DS), jnp.float32),
            pltpu.VMEM((CH * L,), jnp.float32),
            pltpu.SemaphoreType.DMA,
            pltpu.SemaphoreType.DMA,
            pltpu.VMEM_SHARED((NPAD, DS), jnp.float32),
            pltpu.VMEM_SHARED((NPAD, DS), jnp.float32),
        ],
        compiler_params=pltpu.CompilerParams(use_tc_tiling_on_sc=False),
    )
    return k(*g_halves, row2d, col2d, w2d)


def _dinv_of(deg_ref):
    deg = deg_ref[0] + deg_ref[1] + 1.0
    return jnp.where(deg > 0, lax.rsqrt(deg), 0.0)


def _tc1_body(deg_ref, x_ref, w1_ref, glo_ref, ghi_ref):
    dinv = _dinv_of(deg_ref)
    z = jnp.dot(x_ref[...], w1_ref[...], preferred_element_type=jnp.float32)
    g = z * dinv[:, None]
    glo_ref[...] = g[:, :DH // 2]
    ghi_ref[...] = g[:, DH // 2:]


def _tc2_body(deg_ref, slo_ref, shi_ref, glo_ref, ghi_ref, w2_ref, b1_ref,
              g2_ref):
    dinv = _dinv_of(deg_ref)
    s = jnp.concatenate(
        [slo_ref[0] + slo_ref[1] + glo_ref[...],
         shi_ref[0] + shi_ref[1] + ghi_ref[...]], axis=1)
    agg = s * dinv[:, None]
    h1 = jnp.maximum(agg + b1_ref[...][None, :], 0.0)
    z2 = jnp.dot(h1, w2_ref[...], preferred_element_type=jnp.float32)
    g2_ref[...] = z2 * dinv[:, None]


def _tc3_body(deg_ref, s_ref, g2_ref, b2_ref, out_ref):
    dinv = _dinv_of(deg_ref)
    out_ref[...] = ((s_ref[0] + s_ref[1] + g2_ref[...]) * dinv[:, None]
                    + b2_ref[...][None, :])


def _tc1(degp, xpad, W1):
    return pl.pallas_call(
        _tc1_body,
        out_shape=[jax.ShapeDtypeStruct((NPAD, DH // 2), jnp.float32)] * 2,
    )(degp, xpad, W1)


def _tc2(degp, s1lo, s1hi, g1lo, g1hi, W2, b1):
    return pl.pallas_call(
        _tc2_body,
        out_shape=jax.ShapeDtypeStruct((NPAD, DC), jnp.float32),
    )(degp, s1lo, s1hi, g1lo, g1hi, W2, b1)


def _tc3(degp, s2, g2, b2):
    return pl.pallas_call(
        _tc3_body,
        out_shape=jax.ShapeDtypeStruct((NPAD, DC), jnp.float32),
    )(degp, s2, g2, b2)


def kernel(x, edge_index, edge_weight, W1, b1, W2, b2):
    row2d = jnp.zeros((EPAD,), jnp.int32).at[:E].set(
        edge_index[0].astype(jnp.int32)).reshape(EPAD // CH, CH)
    col2d = jnp.zeros((EPAD,), jnp.int32).at[:E].set(
        edge_index[1].astype(jnp.int32)).reshape(EPAD // CH, CH)
    w2d = jnp.zeros((EPAD,), jnp.float32).at[:E].set(
        edge_weight.astype(jnp.float32)).reshape(EPAD // CH, CH)
    xpad = jnp.zeros((NPAD, DF), jnp.float32).at[:N].set(x)

    degp = _run_deg(col2d, w2d)
    g1lo, g1hi = _tc1(degp, xpad, W1)
    s1lo, s1hi = _run_msg([g1lo, g1hi], row2d, col2d, w2d, DH // 2)
    g2 = _tc2(degp, s1lo, s1hi, g1lo, g1hi, W2, b1)
    s2, = _run_msg([g2], row2d, col2d, w2d, DC)
    out = _tc3(degp, s2, g2, b2)
    return out[:N]
